# baseline (speedup 1.0000x reference)
import functools

import jax
import jax.numpy as jnp
from jax.experimental import pallas as pl

_NC = 21


def _ceil(x, m):
    return (x + m - 1) // m * m


def _conv_body(x_ref, w_ref, b_ref, o_ref, *, h, w, coutp):
    hw = h * w
    c = x_ref.shape[1]
    f32 = jnp.float32
    xv = x_ref[0].reshape(c, hw)
    dhs = (-1, 0, 1) if h > 1 else (0,)
    dws = (-1, 0, 1) if w > 1 else (0,)
    xs = {0: xv}
    if h > 1:
        z = jnp.zeros((c, w), f32)
        xs[-1] = jnp.concatenate([z, xv], axis=1)[:, :hw]
        xs[1] = jnp.concatenate([xv, z], axis=1)[:, w:]

    def mm(k, rhs):
        return jax.lax.dot_general(w_ref[k], rhs, (((1,), (0,)), ((), ())),
                                   preferred_element_type=f32)

    out = None
    for dw in dws:
        acc = None
        for dh in dhs:
            t = mm((dh + 1) * 3 + (dw + 1), xs[dh])
            acc = t if acc is None else acc + t
        if dw != 0:
            lane = jax.lax.broadcasted_iota(jnp.int32, (1, hw), 1)
            zc = jnp.zeros((coutp, 1), f32)
            if dw == -1:
                acc = jnp.where(lane % w != 0,
                                jnp.concatenate([zc, acc], axis=1)[:, :hw], 0.0)
            else:
                acc = jnp.where(lane % w != w - 1,
                                jnp.concatenate([acc, zc], axis=1)[:, 1:], 0.0)
        out = acc if out is None else out + acc
    o_ref[0] = out + b_ref[...]


def _mm_body(x_ref, w_ref, b_ref, o_ref):
    o_ref[...] = jax.lax.dot_general(
        w_ref[...], x_ref[...], (((1,), (0,)), ((), ())),
        preferred_element_type=jnp.float32) + b_ref[...]


def _level(x, reg_w, reg_b, cls_w, cls_b):
    b, c, h, w = x.shape
    hw = h * w
    a = reg_w.shape[0] // 4
    cout = 25 * a
    coutp = _ceil(cout, 8)

    wcat = jnp.concatenate([reg_w, cls_w], axis=0)
    bias = jnp.concatenate([reg_b, cls_b])
    bias = jnp.pad(bias, (0, coutp - cout))[:, None]

    if hw == 1:
        wc = jnp.pad(wcat[:, :, 1, 1], ((0, coutp - cout), (0, 0)))
        xt = jnp.transpose(x.reshape(b, c), (1, 0))
        out = pl.pallas_call(
            _mm_body,
            out_shape=jax.ShapeDtypeStruct((coutp, b), jnp.float32),
        )(xt, wc, bias)
        y = jnp.transpose(out, (1, 0))
        reg = y[:, :4 * a].reshape(b, a, 4)
        cls = y[:, 4 * a:cout].reshape(b, a, _NC)
        return reg, cls

    wk = jnp.transpose(wcat, (2, 3, 0, 1)).reshape(9, cout, c)
    wk = jnp.pad(wk, ((0, 0), (0, coutp - cout), (0, 0)))

    out = pl.pallas_call(
        functools.partial(_conv_body, h=h, w=w, coutp=coutp),
        grid=(b,),
        in_specs=[
            pl.BlockSpec((1, c, h, w), lambda i: (i, 0, 0, 0)),
            pl.BlockSpec((9, coutp, c), lambda i: (0, 0, 0)),
            pl.BlockSpec((coutp, 1), lambda i: (0, 0)),
        ],
        out_specs=pl.BlockSpec((1, coutp, hw), lambda i: (i, 0, 0)),
        out_shape=jax.ShapeDtypeStruct((b, coutp, hw), jnp.float32),
    )(x, wk, bias)

    y = jnp.transpose(out, (0, 2, 1))
    reg = y[..., :4 * a].reshape(b, hw * a, 4)
    cls = y[..., 4 * a:cout].reshape(b, hw * a, _NC)
    return reg, cls


def kernel(x0, x1, x2, x3, x4, x5, reg_w0, reg_b0, cls_w0, cls_b0, reg_w1, reg_b1, cls_w1, cls_b1, reg_w2, reg_b2, cls_w2, cls_b2, reg_w3, reg_b3, cls_w3, cls_b3, reg_w4, reg_b4, cls_w4, cls_b4, reg_w5, reg_b5, cls_w5, cls_b5):
    xs = [x0, x1, x2, x3, x4, x5]
    rws = [reg_w0, reg_w1, reg_w2, reg_w3, reg_w4, reg_w5]
    rbs = [reg_b0, reg_b1, reg_b2, reg_b3, reg_b4, reg_b5]
    cws = [cls_w0, cls_w1, cls_w2, cls_w3, cls_w4, cls_w5]
    cbs = [cls_b0, cls_b1, cls_b2, cls_b3, cls_b4, cls_b5]
    regs, clss = [], []
    for i in range(6):
        r, cl = _level(xs[i], rws[i], rbs[i], cws[i], cbs[i])
        regs.append(r)
        clss.append(cl)
    bbox = jnp.concatenate(regs, axis=1)
    cls = jnp.concatenate(clss, axis=1)
    return (bbox, cls)


# speedup vs baseline: 1.4711x; 1.4711x over previous
import functools

import jax
import jax.numpy as jnp
from jax.experimental import pallas as pl

_NC = 21


def _ceil(x, m):
    return (x + m - 1) // m * m


def _conv_body(x_ref, w_ref, b_ref, o_ref, *, h, w, coutp):
    hw = h * w
    c = x_ref.shape[1]
    f32 = jnp.float32
    xv = x_ref[0]
    dhs = (-1, 0, 1) if h > 1 else (0,)
    dws = (-1, 0, 1) if w > 1 else (0,)
    xs = {0: xv}
    if h > 1:
        z = jnp.zeros((c, w), xv.dtype)
        xs[-1] = jnp.concatenate([z, xv], axis=1)[:, :hw]
        xs[1] = jnp.concatenate([xv, z], axis=1)[:, w:]

    def mm(k, rhs):
        return jax.lax.dot_general(w_ref[k], rhs, (((1,), (0,)), ((), ())),
                                   preferred_element_type=f32)

    out = None
    for dw in dws:
        acc = None
        for dh in dhs:
            t = mm((dh + 1) * 3 + (dw + 1), xs[dh])
            acc = t if acc is None else acc + t
        if dw != 0:
            lane = jax.lax.broadcasted_iota(jnp.int32, (1, hw), 1)
            zc = jnp.zeros((coutp, 1), f32)
            if dw == -1:
                acc = jnp.where(lane % w != 0,
                                jnp.concatenate([zc, acc], axis=1)[:, :hw], 0.0)
            else:
                acc = jnp.where(lane % w != w - 1,
                                jnp.concatenate([acc, zc], axis=1)[:, 1:], 0.0)
        out = acc if out is None else out + acc
    o_ref[0] = out + b_ref[...]


def _mm_body(x_ref, w_ref, b_ref, o_ref):
    o_ref[...] = jax.lax.dot_general(
        w_ref[...], x_ref[...], (((1,), (0,)), ((), ())),
        preferred_element_type=jnp.float32) + b_ref[...]


def _level(x, reg_w, reg_b, cls_w, cls_b):
    b, c, h, w = x.shape
    hw = h * w
    a = reg_w.shape[0] // 4
    cout = 25 * a
    coutp = _ceil(cout, 8)

    wcat = jnp.concatenate([reg_w, cls_w], axis=0)
    bias = jnp.concatenate([reg_b, cls_b])
    bias = jnp.pad(bias, (0, coutp - cout))[:, None]

    if hw == 1:
        wc = jnp.pad(wcat[:, :, 1, 1], ((0, coutp - cout), (0, 0)))
        xt = jnp.transpose(x.reshape(b, c), (1, 0))
        out = pl.pallas_call(
            _mm_body,
            out_shape=jax.ShapeDtypeStruct((coutp, b), jnp.float32),
        )(xt, wc, bias)
        y = jnp.transpose(out, (1, 0))
        reg = y[:, :4 * a].reshape(b, a, 4)
        cls = y[:, 4 * a:cout].reshape(b, a, _NC)
        return reg, cls

    wk = jnp.transpose(wcat, (2, 3, 0, 1)).reshape(9, cout, c)
    wk = jnp.pad(wk, ((0, 0), (0, coutp - cout), (0, 0)))
    wk = wk.astype(jnp.bfloat16)

    xf = x.astype(jnp.bfloat16).reshape(b, c, hw)
    out = pl.pallas_call(
        functools.partial(_conv_body, h=h, w=w, coutp=coutp),
        grid=(b,),
        in_specs=[
            pl.BlockSpec((1, c, hw), lambda i: (i, 0, 0)),
            pl.BlockSpec((9, coutp, c), lambda i: (0, 0, 0)),
            pl.BlockSpec((coutp, 1), lambda i: (0, 0)),
        ],
        out_specs=pl.BlockSpec((1, coutp, hw), lambda i: (i, 0, 0)),
        out_shape=jax.ShapeDtypeStruct((b, coutp, hw), jnp.float32),
    )(xf, wk, bias)

    y = jnp.transpose(out, (0, 2, 1))
    reg = y[..., :4 * a].reshape(b, hw * a, 4)
    cls = y[..., 4 * a:cout].reshape(b, hw * a, _NC)
    return reg, cls


def kernel(x0, x1, x2, x3, x4, x5, reg_w0, reg_b0, cls_w0, cls_b0, reg_w1, reg_b1, cls_w1, cls_b1, reg_w2, reg_b2, cls_w2, cls_b2, reg_w3, reg_b3, cls_w3, cls_b3, reg_w4, reg_b4, cls_w4, cls_b4, reg_w5, reg_b5, cls_w5, cls_b5):
    xs = [x0, x1, x2, x3, x4, x5]
    rws = [reg_w0, reg_w1, reg_w2, reg_w3, reg_w4, reg_w5]
    rbs = [reg_b0, reg_b1, reg_b2, reg_b3, reg_b4, reg_b5]
    cws = [cls_w0, cls_w1, cls_w2, cls_w3, cls_w4, cls_w5]
    cbs = [cls_b0, cls_b1, cls_b2, cls_b3, cls_b4, cls_b5]
    regs, clss = [], []
    for i in range(6):
        r, cl = _level(xs[i], rws[i], rbs[i], cws[i], cbs[i])
        regs.append(r)
        clss.append(cl)
    bbox = jnp.concatenate(regs, axis=1)
    cls = jnp.concatenate(clss, axis=1)
    return (bbox, cls)


# speedup vs baseline: 1.5222x; 1.0348x over previous
import functools

import jax
import jax.numpy as jnp
from jax.experimental import pallas as pl

_NC = 21


def _ceil(x, m):
    return (x + m - 1) // m * m


_CHUNK = 2048


def _conv_body(x_ref, w_ref, b_ref, o_ref, *, h, w, nb, base):
    rows = h * w * nb
    f32 = jnp.float32
    for s in range(0, rows, _CHUNK):
        n = min(_CHUNK, rows - s)

        def mm(k, s=s, n=n):
            start = base + s + ((k // 3 - 1) * w + (k % 3 - 1)) * nb
            return jax.lax.dot_general(
                x_ref[start:start + n, :], w_ref[k],
                (((1,), (0,)), ((), ())), preferred_element_type=f32)

        col = ((jax.lax.broadcasted_iota(jnp.int32, (n, 1), 0) + s) // nb) % w
        out = None
        for dw in (-1, 0, 1):
            acc = None
            for dh in (-1, 0, 1):
                t = mm((dh + 1) * 3 + (dw + 1))
                acc = t if acc is None else acc + t
            if dw == -1:
                acc = jnp.where(col != 0, acc, 0.0)
            elif dw == 1:
                acc = jnp.where(col != w - 1, acc, 0.0)
            out = acc if out is None else out + acc
        o_ref[s:s + n, :] = out + b_ref[...]


def _mm_body(x_ref, w_ref, b_ref, o_ref):
    o_ref[...] = jax.lax.dot_general(
        x_ref[...], w_ref[...], (((1,), (0,)), ((), ())),
        preferred_element_type=jnp.float32) + b_ref[...]


def _level(x, reg_w, reg_b, cls_w, cls_b):
    nb, c, h, w = x.shape
    a = reg_w.shape[0] // 4
    cout = 25 * a
    coutp = _ceil(cout, 8)

    wcat = jnp.concatenate([reg_w, cls_w], axis=0)
    bias = jnp.concatenate([reg_b, cls_b])
    bias = jnp.pad(bias, (0, coutp - cout))[None, :]

    if h * w == 1:
        wc = jnp.pad(wcat[:, :, 1, 1], ((0, coutp - cout), (0, 0)))
        wc = jnp.transpose(wc, (1, 0)).astype(jnp.bfloat16)
        xb = x.reshape(nb, c).astype(jnp.bfloat16)
        out = pl.pallas_call(
            _mm_body,
            out_shape=jax.ShapeDtypeStruct((nb, coutp), jnp.float32),
        )(xb, wc, bias)
        reg = out[:, :4 * a].reshape(nb, a, 4)
        cls = out[:, 4 * a:cout].reshape(nb, a, _NC)
        return reg, cls

    wk = jnp.transpose(wcat, (2, 3, 1, 0)).reshape(9, c, cout)
    wk = jnp.pad(wk, ((0, 0), (0, 0), (0, coutp - cout)))
    wk = wk.astype(jnp.bfloat16)

    rows = h * w * nb
    base = nb * (w + 1)
    xt = jnp.transpose(x, (2, 3, 0, 1)).reshape(rows, c).astype(jnp.bfloat16)
    xp = jnp.pad(xt, ((base, base), (0, 0)))

    out = pl.pallas_call(
        functools.partial(_conv_body, h=h, w=w, nb=nb, base=base),
        out_shape=jax.ShapeDtypeStruct((rows, coutp), jnp.float32),
    )(xp, wk, bias)

    y = out.reshape(h, w, nb, coutp)
    reg = y[..., :4 * a].transpose(2, 0, 1, 3).reshape(nb, h * w * a, 4)
    cls = y[..., 4 * a:cout].transpose(2, 0, 1, 3).reshape(nb, h * w * a, _NC)
    return reg, cls


def kernel(x0, x1, x2, x3, x4, x5, reg_w0, reg_b0, cls_w0, cls_b0, reg_w1, reg_b1, cls_w1, cls_b1, reg_w2, reg_b2, cls_w2, cls_b2, reg_w3, reg_b3, cls_w3, cls_b3, reg_w4, reg_b4, cls_w4, cls_b4, reg_w5, reg_b5, cls_w5, cls_b5):
    xs = [x0, x1, x2, x3, x4, x5]
    rws = [reg_w0, reg_w1, reg_w2, reg_w3, reg_w4, reg_w5]
    rbs = [reg_b0, reg_b1, reg_b2, reg_b3, reg_b4, reg_b5]
    cws = [cls_w0, cls_w1, cls_w2, cls_w3, cls_w4, cls_w5]
    cbs = [cls_b0, cls_b1, cls_b2, cls_b3, cls_b4, cls_b5]
    regs, clss = [], []
    for i in range(6):
        r, cl = _level(xs[i], rws[i], rbs[i], cws[i], cbs[i])
        regs.append(r)
        clss.append(cl)
    bbox = jnp.concatenate(regs, axis=1)
    cls = jnp.concatenate(clss, axis=1)
    return (bbox, cls)
